# Initial kernel scaffold; baseline (speedup 1.0000x reference)
import jax
import jax.numpy as jnp
from jax.experimental import pallas as pl


def kernel(input, edge_index, edge_weight, W, bias):
    raise NotImplementedError("write your pallas kernel here")


# speedup vs baseline: 5.1464x; 5.1464x over previous
import functools

import jax
import jax.numpy as jnp
from jax import lax
from jax.experimental import pallas as pl
from jax.experimental.pallas import tpu as pltpu
from jax.experimental.pallas import tpu_sc as plsc

NC = 2
NS = 16
L = 16
NW = NC * NS
K = 128


def _sc_spmm(n, d, ep):
    ept = ep // NW
    nchunk = ept // K
    ng = K // L
    dl = d // L
    rpt = (n // (8 * NS)) * 8
    tail = n - NS * rpt

    mesh = plsc.VectorSubcoreMesh(core_axis_name="c", subcore_axis_name="s")

    @functools.partial(
        pl.kernel,
        out_type=jax.ShapeDtypeStruct((NC, n, d), jnp.float32),
        mesh=mesh,
        scratch_types=[
            pltpu.VMEM((nchunk, K), jnp.int32),
            pltpu.VMEM((nchunk, K), jnp.int32),
            pltpu.VMEM((ept,), jnp.float32),
            pltpu.VMEM((K, d), jnp.float32),
            pltpu.VMEM_SHARED((n, d), jnp.float32),
            pltpu.SemaphoreType.DMA,
        ],
    )
    def sc_kernel(x_hbm, src_hbm, dst_hbm, w_hbm, out_hbm,
                  src_v, dst_v, w_v, rows_v, acc, sem):
        c = lax.axis_index("c")
        s = lax.axis_index("s")
        wid = c * NS + s
        ebase = pl.multiple_of(wid * ept, 8)

        pltpu.sync_copy(src_hbm.at[wid], src_v)
        pltpu.sync_copy(dst_hbm.at[wid], dst_v)
        pltpu.sync_copy(w_hbm.at[pl.ds(ebase, ept)], w_v)

        zvec = jnp.zeros((L,), jnp.float32)

        @pl.loop(0, K)
        def _zrow(r):
            for t in range(dl):
                rows_v[r, pl.ds(t * L, L)] = zvec

        nzfull = rpt // K
        zrem = rpt - nzfull * K
        for b in range(nzfull):
            zoff = pl.multiple_of(s * rpt + b * K, 8)
            pltpu.sync_copy(rows_v, acc.at[pl.ds(zoff, K)])
        if zrem:
            zoff = pl.multiple_of(s * rpt + nzfull * K, 8)
            pltpu.sync_copy(rows_v.at[pl.ds(0, zrem)], acc.at[pl.ds(zoff, zrem)])
        if tail:
            @pl.when(s == NS - 1)
            def _ztail():
                zoff = pl.multiple_of(NS * rpt, 8)
                pltpu.sync_copy(rows_v.at[pl.ds(0, tail)],
                                acc.at[pl.ds(zoff, tail)])
        plsc.subcore_barrier()

        @pl.loop(0, nchunk)
        def _chunk(j):
            pltpu.async_copy(x_hbm.at[src_v.at[j]], rows_v, sem).wait()

            @pl.loop(0, ng)
            def _group(g):
                w16 = w_v[pl.ds(j * K + g * L, L)]
                for r in range(L):
                    wv = jnp.full((L,), w16[r])
                    for t in range(dl):
                        rows_v[g * L + r, pl.ds(t * L, L)] = (
                            rows_v[g * L + r, pl.ds(t * L, L)] * wv)

            pltpu.sync_copy(rows_v, acc.at[dst_v.at[j]], add=True)

        plsc.subcore_barrier()

        ob = pl.multiple_of(s * rpt, 8)
        pltpu.sync_copy(acc.at[pl.ds(ob, rpt)],
                        out_hbm.at[c, pl.ds(ob, rpt)])
        if tail:
            @pl.when(s == NS - 1)
            def _copy_tail():
                tb = pl.multiple_of(NS * rpt, 8)
                pltpu.sync_copy(acc.at[pl.ds(tb, tail)],
                                out_hbm.at[c, pl.ds(tb, tail)])

    return sc_kernel


def _tc_body(p0_ref, p1_ref, w_ref, b_ref, o_ref):
    agg = p0_ref[...] + p1_ref[...]
    o_ref[...] = (
        jnp.dot(agg, w_ref[...], preferred_element_type=jnp.float32)
        + b_ref[...])


def kernel(input, edge_index, edge_weight, W, bias):
    n, d_in = input.shape
    d_out = W.shape[1]
    e = edge_index.shape[1]
    ep = ((e + NW * K - 1) // (NW * K)) * NW * K
    nchunk = ep // (NW * K)

    pad = ep - e
    dst = jnp.pad(edge_index[0], (0, pad)).reshape(NW, nchunk, K)
    src = jnp.pad(edge_index[1], (0, pad)).reshape(NW, nchunk, K)
    w = jnp.pad(edge_weight, (0, pad))

    partials = _sc_spmm(n, d_in, ep)(input, src, dst, w)

    nb = 10
    bn = n // nb
    out = pl.pallas_call(
        _tc_body,
        grid=(nb,),
        in_specs=[
            pl.BlockSpec((bn, d_in), lambda i: (i, 0)),
            pl.BlockSpec((bn, d_in), lambda i: (i, 0)),
            pl.BlockSpec((d_in, d_out), lambda i: (0, 0)),
            pl.BlockSpec((1, d_out), lambda i: (0, 0)),
        ],
        out_specs=pl.BlockSpec((bn, d_out), lambda i: (i, 0)),
        out_shape=jax.ShapeDtypeStruct((n, d_out), jnp.float32),
    )(partials[0], partials[1], W, bias)
    return out


# speedup vs baseline: 10.9102x; 2.1200x over previous
import functools

import jax
import jax.numpy as jnp
from jax import lax
from jax.experimental import pallas as pl
from jax.experimental.pallas import tpu as pltpu
from jax.experimental.pallas import tpu_sc as plsc

NC = 2
NS = 16
L = 16
NW = NC * NS
K = 128
GC = 10


def _sc_spmm(n, d, ngroup):
    ng = K // L
    dl = d // L
    rpt = (n // (8 * NS)) * 8
    tail = n - NS * rpt

    mesh = plsc.VectorSubcoreMesh(core_axis_name="c", subcore_axis_name="s")

    @functools.partial(
        pl.kernel,
        out_type=jax.ShapeDtypeStruct((NC, n, d), jnp.float32),
        mesh=mesh,
        scratch_types=[
            pltpu.VMEM((2, GC * 2, K), jnp.int32),
            pltpu.VMEM((2, GC, K), jnp.float32),
            pltpu.VMEM((K, d), jnp.float32),
            pltpu.VMEM((K, d), jnp.float32),
            pltpu.VMEM_SHARED((n, d), jnp.float32),
            pltpu.SemaphoreType.DMA,
            pltpu.SemaphoreType.DMA,
            pltpu.SemaphoreType.DMA,
            pltpu.SemaphoreType.DMA,
        ],
    )
    def sc_kernel(x_hbm, ed_hbm, wf_hbm, out_hbm,
                  ebuf, wbuf, rows_a, rows_b, acc, sem_a, sem_b, sem_g,
                  sem_w):
        c = lax.axis_index("c")
        s = lax.axis_index("s")
        wid = c * NS + s

        zvec = jnp.zeros((L,), jnp.float32)

        @pl.loop(0, K)
        def _zrow(r):
            for t in range(dl):
                rows_a[r, pl.ds(t * L, L)] = zvec

        nzfull = rpt // K
        zrem = rpt - nzfull * K
        for b in range(nzfull):
            zoff = pl.multiple_of(s * rpt + b * K, 8)
            pltpu.sync_copy(rows_a, acc.at[pl.ds(zoff, K)])
        if zrem:
            zoff = pl.multiple_of(s * rpt + nzfull * K, 8)
            pltpu.sync_copy(rows_a.at[pl.ds(0, zrem)],
                            acc.at[pl.ds(zoff, zrem)])
        if tail:
            @pl.when(s == NS - 1)
            def _ztail():
                zoff = pl.multiple_of(NS * rpt, 8)
                pltpu.sync_copy(rows_a.at[pl.ds(0, tail)],
                                acc.at[pl.ds(zoff, tail)])
        plsc.subcore_barrier()

        pltpu.sync_copy(ed_hbm.at[wid, 0], ebuf.at[0])
        pltpu.sync_copy(wf_hbm.at[wid, 0], wbuf.at[0])
        pltpu.async_copy(x_hbm.at[ebuf.at[0, 0]], rows_a, sem_a)

        def scale(rows_ref, p, t):
            @pl.loop(0, ng)
            def _g16(g):
                w16 = wbuf[p, t, pl.ds(g * L, L)]
                for r in range(L):
                    wv = jnp.full((L,), w16[r])
                    for u in range(dl):
                        rows_ref[g * L + r, pl.ds(u * L, L)] = (
                            rows_ref[g * L + r, pl.ds(u * L, L)] * wv)

        @pl.loop(0, ngroup)
        def _group(gi):
            p = lax.rem(gi, 2)
            pn = 1 - p

            @pl.when(gi < ngroup - 1)
            def _pref():
                pltpu.async_copy(ed_hbm.at[wid, gi + 1], ebuf.at[pn], sem_g)
                pltpu.async_copy(wf_hbm.at[wid, gi + 1], wbuf.at[pn], sem_w)

            for t in range(GC):
                cur, csem = (rows_a, sem_a) if t % 2 == 0 else (rows_b, sem_b)
                nxt, nsem = (rows_b, sem_b) if t % 2 == 0 else (rows_a, sem_a)

                if t < GC - 1:
                    pltpu.async_copy(
                        x_hbm.at[ebuf.at[p, 2 * (t + 1)]], nxt, nsem)
                else:
                    @pl.when(gi < ngroup - 1)
                    def _next_group_gather():
                        pltpu.make_async_copy(
                            ed_hbm.at[wid, gi + 1], ebuf.at[pn], sem_g).wait()
                        pltpu.make_async_copy(
                            wf_hbm.at[wid, gi + 1], wbuf.at[pn], sem_w).wait()
                        pltpu.async_copy(x_hbm.at[ebuf.at[pn, 0]], nxt, nsem)

                pltpu.make_async_copy(
                    x_hbm.at[ebuf.at[p, 2 * t]], cur, csem).wait()
                scale(cur, p, t)
                pltpu.sync_copy(cur, acc.at[ebuf.at[p, 2 * t + 1]], add=True)

        plsc.subcore_barrier()

        ob = pl.multiple_of(s * rpt, 8)
        pltpu.sync_copy(acc.at[pl.ds(ob, rpt)],
                        out_hbm.at[c, pl.ds(ob, rpt)])
        if tail:
            @pl.when(s == NS - 1)
            def _copy_tail():
                tb = pl.multiple_of(NS * rpt, 8)
                pltpu.sync_copy(acc.at[pl.ds(tb, tail)],
                                out_hbm.at[c, pl.ds(tb, tail)])

    return sc_kernel


def _tc_body(p0_ref, p1_ref, w_ref, b_ref, o_ref):
    agg = p0_ref[...] + p1_ref[...]
    o_ref[...] = (
        jnp.dot(agg, w_ref[...], preferred_element_type=jnp.float32)
        + b_ref[...])


def kernel(input, edge_index, edge_weight, W, bias):
    n, d_in = input.shape
    d_out = W.shape[1]
    e = edge_index.shape[1]
    egrp = NW * K * GC
    ep = ((e + egrp - 1) // egrp) * egrp
    ngroup = ep // (NW * K * GC)
    pad = ep - e

    spread = jnp.arange(pad, dtype=jnp.int32) % n
    dst = jnp.concatenate([edge_index[0], spread])
    src = jnp.concatenate([edge_index[1], spread])
    wf = jnp.concatenate([edge_weight, jnp.zeros((pad,), jnp.float32)]
                         ).reshape(NW, ngroup, GC, K)

    ed = jnp.stack(
        [x.reshape(NW, ngroup, GC, 1, K) for x in (src, dst)], axis=3
    ).reshape(NW, ngroup, GC * 2, K)

    partials = _sc_spmm(n, d_in, ngroup)(input, ed, wf)

    nb = 10
    bn = n // nb
    out = pl.pallas_call(
        _tc_body,
        grid=(nb,),
        in_specs=[
            pl.BlockSpec((bn, d_in), lambda i: (i, 0)),
            pl.BlockSpec((bn, d_in), lambda i: (i, 0)),
            pl.BlockSpec((d_in, d_out), lambda i: (0, 0)),
            pl.BlockSpec((1, d_out), lambda i: (0, 0)),
        ],
        out_specs=pl.BlockSpec((bn, d_out), lambda i: (i, 0)),
        out_shape=jax.ShapeDtypeStruct((n, d_out), jnp.float32),
    )(partials[0], partials[1], W, bias)
    return out
